# baseline (speedup 1.0000x reference)
import jax
import jax.numpy as jnp
from jax.experimental import pallas as pl
from jax.experimental.pallas import tpu as pltpu

DIM = 384
E = 8
TOPK = 2
H = 7
W = 7
B = 16
DSTATE = 16
DTRANK = 48
L = H * W
NXP = DTRANK + 2 * DSTATE
TPB = 8
CAP = 20.0
EPS = 1e-6
_HIGHEST = jax.lax.Precision.HIGHEST


def _softplus(x):
    return jnp.where(x > 0.0, x, 0.0) + jnp.log1p(jnp.exp(-jnp.abs(x)))


def _ln_rows(x, g, b, eps=1e-5):
    m = jnp.mean(x, axis=-1, keepdims=True)
    v = jnp.mean((x - m) ** 2, axis=-1, keepdims=True)
    return (x - m) * jax.lax.rsqrt(v + eps) * g + b


def _top1(vals, iota):
    m = jnp.max(vals, axis=1, keepdims=True)
    idx = jnp.min(jnp.where(vals == m, iota, E), axis=1, keepdims=True)
    return m, idx


def _gate_body(x_ref, Wg_ref, bg_ref, sval_ref, sidx_ref, aux_ref):
    xs = x_ref[...]
    xf = jnp.mean(xs, axis=(1, 2))
    logits = jax.lax.dot_general(xf, Wg_ref[...], (((1,), (0,)), ((), ())),
                                 precision=_HIGHEST) + bg_ref[...]
    z = logits - jnp.max(logits, axis=1, keepdims=True)
    ez = jnp.exp(z)
    raw = ez / jnp.sum(ez, axis=1, keepdims=True)

    iota = jax.lax.broadcasted_iota(jnp.int32, (B, E), 1)
    _, i1 = _top1(raw, iota)
    mask1 = iota == i1
    _, i2 = _top1(jnp.where(mask1, -1.0, raw), iota)
    mask = jnp.logical_or(mask1, iota == i2).astype(jnp.float32)

    masked = raw * mask
    denom = jnp.sum(masked, axis=0, keepdims=True) + EPS
    gs = masked / denom * CAP

    importance = jnp.mean(raw, axis=0, keepdims=True)
    load = jnp.mean(mask, axis=0, keepdims=True)
    aux_ref[...] = 0.01 * jnp.mean((load - importance) ** 2,
                                   axis=(0, 1), keepdims=True)

    g1v, gi1 = _top1(gs, iota)
    g2v, gi2 = _top1(jnp.where(iota == gi1, -1.0, gs), iota)
    sval_ref[...] = jnp.concatenate([g1v, g2v], axis=1)
    sidx_ref[...] = jnp.concatenate([gi1, gi2], axis=1).astype(jnp.int32)


def _expert_body(sidx_ref, sval_ref, x_ref,
                 Wx_ref, Wdt_ref, bdt_ref, Alog_ref, Dexp_ref,
                 g1_ref, b1_ref, g2_ref, b2_ref,
                 out_ref, da_ref, dbu_ref, hsf_ref, hsb_ref):
    bb = pl.program_id(0)

    pio = jax.lax.broadcasted_iota(jnp.int32, (L, L), 0)
    qio = jax.lax.broadcasted_iota(jnp.int32, (L, L), 1)
    T = jnp.logical_and(pio // W == qio % H, pio % W == qio // H).astype(jnp.float32)

    for tt in range(TPB):
        _one_token(bb * TPB + tt, tt, T,
                   sidx_ref, sval_ref, x_ref,
                   Wx_ref, Wdt_ref, bdt_ref, Alog_ref, Dexp_ref,
                   g1_ref, b1_ref, g2_ref, b2_ref,
                   out_ref, da_ref, dbu_ref, hsf_ref, hsb_ref)


def _one_token(b, tt, T, sidx_ref, sval_ref, x_ref,
               Wx_ref, Wdt_ref, bdt_ref, Alog_ref, Dexp_ref,
               g1_ref, b1_ref, g2_ref, b2_ref,
               out_ref, da_ref, dbu_ref, hsf_ref, hsb_ref):
    e0 = sidx_ref[b, 0]
    e1 = sidx_ref[b, 1]
    Wx = (Wx_ref[e0], Wx_ref[e1])
    Wdt = (Wdt_ref[e0], Wdt_ref[e1])
    bdt = (bdt_ref[e0], bdt_ref[e1])
    Alog = (Alog_ref[e0], Alog_ref[e1])
    Dexp = (Dexp_ref[e0], Dexp_ref[e1])
    g1 = (g1_ref[e0], g1_ref[e1])
    b1 = (b1_ref[e0], b1_ref[e1])
    g2 = (g2_ref[e0], g2_ref[e1])
    b2 = (b2_ref[e0], b2_ref[e1])

    seq_h = x_ref[tt].reshape(L, DIM)
    seq_v = jax.lax.dot_general(T, seq_h, (((1,), (0,)), ((), ())),
                                precision=_HIGHEST)
    seq2 = jnp.concatenate([seq_h, seq_v], axis=0)

    Wcat = jnp.concatenate([Wx[0], Wx[1]], axis=1)
    xd2 = jax.lax.dot_general(seq2, Wcat, (((1,), (0,)), ((), ())),
                              precision=_HIGHEST)

    Cms = []
    for j in range(2):
        xd = xd2[:, j * NXP:(j + 1) * NXP]
        delta2 = _softplus(
            jax.lax.dot_general(xd[:, :DTRANK], Wdt[j],
                                (((1,), (0,)), ((), ())),
                                precision=_HIGHEST) + bdt[j])
        Bm2 = xd[:, DTRANK:DTRANK + DSTATE]
        Cms.append(xd[:, DTRANK + DSTATE:])
        AT = -jnp.exp(Alog[j]).T
        du2 = delta2 * seq2
        for o in range(2):
            Br = Bm2[o * L:(o + 1) * L][:, :, None]
            for c in range(7):
                s = slice(7 * c, 7 * c + 7)
                g = slice(o * L + 7 * c, o * L + 7 * c + 7)
                da_ref[j, o, s] = jnp.exp(delta2[g][:, None, :] * AT[None])
                dbu_ref[j, o, s] = du2[g][:, None, :] * Br[s]

    hf = [[jnp.zeros((DSTATE, DIM), jnp.float32) for _ in range(2)]
          for _ in range(2)]
    hb = [[jnp.zeros((DSTATE, DIM), jnp.float32) for _ in range(2)]
          for _ in range(2)]
    for t in range(L):
        r = L - 1 - t
        for j in range(2):
            for o in range(2):
                hf[j][o] = da_ref[j, o, t] * hf[j][o] + dbu_ref[j, o, t]
                hb[j][o] = da_ref[j, o, r] * hb[j][o] + dbu_ref[j, o, r]
                hsf_ref[j, o, t] = hf[j][o]
                hsb_ref[j, o, r] = hb[j][o]

    outsum = None
    yvs = []
    yhs = []
    for j in range(2):
        Cr = Cms[j][:, :, None]
        yos = []
        for o in range(2):
            chunks = []
            for c in range(7):
                s = slice(7 * c, 7 * c + 7)
                hsum = hsf_ref[j, o, s] + hsb_ref[j, o, s]
                chunks.append(jnp.sum(hsum * Cr[o * L + 7 * c:
                                                o * L + 7 * c + 7], axis=1))
            yos.append(jnp.concatenate(chunks, axis=0))
        yhs.append(yos[0])
        yvs.append(yos[1])

    yv_cat = jnp.concatenate(yvs, axis=1)
    yv_un = jax.lax.dot_general(T, yv_cat, (((1,), (0,)), ((), ())),
                                precision=_HIGHEST)

    for j in range(2):
        y = yhs[j] + yv_un[:, j * DIM:(j + 1) * DIM] \
            + 4.0 * seq_h * Dexp[j]
        y = _ln_rows(y, g1[j], b1[j])
        pooled = jnp.mean(y, axis=0, keepdims=True)
        outv = _ln_rows(pooled, g2[j], b2[j])
        contrib = sval_ref[b, j] * outv
        outsum = contrib if outsum is None else outsum + contrib

    out_ref[tt] = outsum


@jax.jit
def kernel(x, Wg, bg, Wx, Wdt, bdt, A_log, Dexp, g1, b1, g2, b2):
    sval, sidx, aux = pl.pallas_call(
        _gate_body,
        out_shape=[
            jax.ShapeDtypeStruct((B, TOPK), jnp.float32),
            jax.ShapeDtypeStruct((B, TOPK), jnp.int32),
            jax.ShapeDtypeStruct((1, 1), jnp.float32),
        ],
    )(x, Wg, bg.reshape(1, E))

    def full(shape):
        nd = len(shape)
        return pl.BlockSpec(shape, lambda b, si, sv, _nd=nd: (0,) * _nd)

    grid_spec = pltpu.PrefetchScalarGridSpec(
        num_scalar_prefetch=2,
        grid=(B // TPB,),
        in_specs=[
            pl.BlockSpec((TPB, H, W, DIM), lambda b, si, sv: (b, 0, 0, 0)),
            full((E, DIM, NXP)),
            full((E, DTRANK, DIM)),
            full((E, 1, DIM)),
            full((E, DIM, DSTATE)),
            full((E, 1, DIM)),
            full((E, 1, DIM)),
            full((E, 1, DIM)),
            full((E, 1, DIM)),
            full((E, 1, DIM)),
        ],
        out_specs=pl.BlockSpec((TPB, 1, DIM), lambda b, si, sv: (b, 0, 0)),
        scratch_shapes=[
            pltpu.VMEM((2, 2, L, DSTATE, DIM), jnp.float32),
            pltpu.VMEM((2, 2, L, DSTATE, DIM), jnp.float32),
            pltpu.VMEM((2, 2, L, DSTATE, DIM), jnp.float32),
            pltpu.VMEM((2, 2, L, DSTATE, DIM), jnp.float32),
        ],
    )

    r3 = lambda a: a.reshape(E, 1, DIM)
    mixed = pl.pallas_call(
        _expert_body,
        grid_spec=grid_spec,
        out_shape=jax.ShapeDtypeStruct((B, 1, DIM), jnp.float32),
        compiler_params=pltpu.CompilerParams(
            dimension_semantics=("arbitrary",),
        ),
    )(sidx, sval, x, Wx, Wdt, r3(bdt), A_log, r3(Dexp),
      r3(g1), r3(b1), r3(g2), r3(b2))

    return mixed.reshape(B, DIM), aux[0, 0]


# speedup vs baseline: 1.2455x; 1.2455x over previous
import jax
import jax.numpy as jnp
from jax.experimental import pallas as pl
from jax.experimental.pallas import tpu as pltpu

DIM = 384
E = 8
TOPK = 2
H = 7
W = 7
B = 16
DSTATE = 16
DTRANK = 48
L = H * W
NXP = DTRANK + 2 * DSTATE
TPB = 4
CAP = 20.0
EPS = 1e-6
_HIGHEST = jax.lax.Precision.HIGHEST


def _softplus(x):
    return jnp.where(x > 0.0, x, 0.0) + jnp.log1p(jnp.exp(-jnp.abs(x)))


def _ln_rows(x, g, b, eps=1e-5):
    m = jnp.mean(x, axis=-1, keepdims=True)
    v = jnp.mean((x - m) ** 2, axis=-1, keepdims=True)
    return (x - m) * jax.lax.rsqrt(v + eps) * g + b


def _top1(vals, iota):
    m = jnp.max(vals, axis=1, keepdims=True)
    idx = jnp.min(jnp.where(vals == m, iota, E), axis=1, keepdims=True)
    return m, idx


def _gate_body(x_ref, Wg_ref, bg_ref, sval_ref, sidx_ref, aux_ref):
    xs = x_ref[...]
    xf = jnp.mean(xs, axis=(1, 2))
    logits = jax.lax.dot_general(xf, Wg_ref[...], (((1,), (0,)), ((), ())),
                                 precision=_HIGHEST) + bg_ref[...]
    z = logits - jnp.max(logits, axis=1, keepdims=True)
    ez = jnp.exp(z)
    raw = ez / jnp.sum(ez, axis=1, keepdims=True)

    iota = jax.lax.broadcasted_iota(jnp.int32, (B, E), 1)
    _, i1 = _top1(raw, iota)
    mask1 = iota == i1
    _, i2 = _top1(jnp.where(mask1, -1.0, raw), iota)
    mask = jnp.logical_or(mask1, iota == i2).astype(jnp.float32)

    masked = raw * mask
    denom = jnp.sum(masked, axis=0, keepdims=True) + EPS
    gs = masked / denom * CAP

    importance = jnp.mean(raw, axis=0, keepdims=True)
    load = jnp.mean(mask, axis=0, keepdims=True)
    aux_ref[...] = 0.01 * jnp.mean((load - importance) ** 2,
                                   axis=(0, 1), keepdims=True)

    g1v, gi1 = _top1(gs, iota)
    g2v, gi2 = _top1(jnp.where(iota == gi1, -1.0, gs), iota)
    sval_ref[...] = jnp.concatenate([g1v, g2v], axis=1)
    sidx_ref[...] = jnp.concatenate([gi1, gi2], axis=1).astype(jnp.int32)


def _expert_body(sidx_ref, sval_ref, x_ref,
                 Wx_ref, Wdt_ref, bdt_ref, Alog_ref, Dexp_ref,
                 g1_ref, b1_ref, g2_ref, b2_ref,
                 out_ref, da_ref, dbu_ref, hsf_ref, hsb_ref, cs_ref, y_ref):
    bb = pl.program_id(0)

    pio = jax.lax.broadcasted_iota(jnp.int32, (L, L), 0)
    qio = jax.lax.broadcasted_iota(jnp.int32, (L, L), 1)
    T = jnp.logical_and(pio // W == qio % H, pio % W == qio // H).astype(jnp.float32)

    for tt in range(TPB):
        _one_token(bb * TPB + tt, tt, T,
                   sidx_ref, sval_ref, x_ref,
                   Wx_ref, Wdt_ref, bdt_ref, Alog_ref, Dexp_ref,
                   g1_ref, b1_ref, g2_ref, b2_ref,
                   out_ref, da_ref, dbu_ref, hsf_ref, hsb_ref, cs_ref, y_ref)


def _one_token(b, tt, T, sidx_ref, sval_ref, x_ref,
               Wx_ref, Wdt_ref, bdt_ref, Alog_ref, Dexp_ref,
               g1_ref, b1_ref, g2_ref, b2_ref,
               out_ref, da_ref, dbu_ref, hsf_ref, hsb_ref, cs_ref, y_ref):
    e0 = sidx_ref[b, 0]
    e1 = sidx_ref[b, 1]
    Wx = (Wx_ref[e0], Wx_ref[e1])
    Wdt = (Wdt_ref[e0], Wdt_ref[e1])
    bdt = (bdt_ref[e0], bdt_ref[e1])
    Alog = (Alog_ref[e0], Alog_ref[e1])
    Dexp = (Dexp_ref[e0], Dexp_ref[e1])
    g1 = (g1_ref[e0], g1_ref[e1])
    b1 = (b1_ref[e0], b1_ref[e1])
    g2 = (g2_ref[e0], g2_ref[e1])
    b2 = (b2_ref[e0], b2_ref[e1])

    seq_h = x_ref[tt].reshape(L, DIM)
    seq_v = jax.lax.dot_general(T, seq_h, (((1,), (0,)), ((), ())),
                                precision=_HIGHEST)
    seq2 = jnp.concatenate([seq_h, seq_v], axis=0)

    Wcat = jnp.concatenate([Wx[0], Wx[1]], axis=1)
    xd2 = jax.lax.dot_general(seq2, Wcat, (((1,), (0,)), ((), ())),
                              precision=_HIGHEST)

    for j in range(2):
        xd = xd2[:, j * NXP:(j + 1) * NXP]
        delta2 = _softplus(
            jax.lax.dot_general(xd[:, :DTRANK], Wdt[j],
                                (((1,), (0,)), ((), ())),
                                precision=_HIGHEST) + bdt[j])
        Bm2 = xd[:, DTRANK:DTRANK + DSTATE]
        cs_ref[j] = xd[:, DTRANK + DSTATE:]
        AT = -jnp.exp(Alog[j]).T
        du2 = delta2 * seq2
        for o in range(2):
            Br = Bm2[o * L:(o + 1) * L][:, :, None]
            for c in range(7):
                s = slice(7 * c, 7 * c + 7)
                g = slice(o * L + 7 * c, o * L + 7 * c + 7)
                da_ref[j, o, s] = jnp.exp(delta2[g][:, None, :] * AT[None])
                dbu_ref[j, o, s] = du2[g][:, None, :] * Br[s]

    for j in range(2):
        for o in range(2):
            hf = jnp.zeros((DSTATE, DIM), jnp.float32)
            hb = jnp.zeros((DSTATE, DIM), jnp.float32)
            for t in range(L):
                r = L - 1 - t
                hf = da_ref[j, o, t] * hf + dbu_ref[j, o, t]
                hb = da_ref[j, o, r] * hb + dbu_ref[j, o, r]
                hsf_ref[j, o, t] = hf
                hsb_ref[j, o, r] = hb

    for j in range(2):
        for o in range(2):
            for c in range(7):
                s = slice(7 * c, 7 * c + 7)
                hsum = hsf_ref[j, o, s] + hsb_ref[j, o, s]
                Cr = cs_ref[j, o * L + 7 * c:o * L + 7 * c + 7, :][:, :, None]
                y_ref[j, o, s] = jnp.sum(hsum * Cr, axis=1)

    yv_cat = jnp.concatenate([y_ref[0, 1], y_ref[1, 1]], axis=1)
    yv_un = jax.lax.dot_general(T, yv_cat, (((1,), (0,)), ((), ())),
                                precision=_HIGHEST)

    outsum = None
    for j in range(2):
        y = y_ref[j, 0] + yv_un[:, j * DIM:(j + 1) * DIM] \
            + 4.0 * seq_h * Dexp[j]
        y = _ln_rows(y, g1[j], b1[j])
        pooled = jnp.mean(y, axis=0, keepdims=True)
        outv = _ln_rows(pooled, g2[j], b2[j])
        contrib = sval_ref[b, j] * outv
        outsum = contrib if outsum is None else outsum + contrib

    out_ref[tt] = outsum


@jax.jit
def kernel(x, Wg, bg, Wx, Wdt, bdt, A_log, Dexp, g1, b1, g2, b2):
    sval, sidx, aux = pl.pallas_call(
        _gate_body,
        out_shape=[
            jax.ShapeDtypeStruct((B, TOPK), jnp.float32),
            jax.ShapeDtypeStruct((B, TOPK), jnp.int32),
            jax.ShapeDtypeStruct((1, 1), jnp.float32),
        ],
    )(x, Wg, bg.reshape(1, E))

    def full(shape):
        nd = len(shape)
        return pl.BlockSpec(shape, lambda b, si, sv, _nd=nd: (0,) * _nd)

    grid_spec = pltpu.PrefetchScalarGridSpec(
        num_scalar_prefetch=2,
        grid=(B // TPB,),
        in_specs=[
            pl.BlockSpec((TPB, H, W, DIM), lambda b, si, sv: (b, 0, 0, 0)),
            full((E, DIM, NXP)),
            full((E, DTRANK, DIM)),
            full((E, 1, DIM)),
            full((E, DIM, DSTATE)),
            full((E, 1, DIM)),
            full((E, 1, DIM)),
            full((E, 1, DIM)),
            full((E, 1, DIM)),
            full((E, 1, DIM)),
        ],
        out_specs=pl.BlockSpec((TPB, 1, DIM), lambda b, si, sv: (b, 0, 0)),
        scratch_shapes=[
            pltpu.VMEM((2, 2, L, DSTATE, DIM), jnp.float32),
            pltpu.VMEM((2, 2, L, DSTATE, DIM), jnp.float32),
            pltpu.VMEM((2, 2, L, DSTATE, DIM), jnp.float32),
            pltpu.VMEM((2, 2, L, DSTATE, DIM), jnp.float32),
            pltpu.VMEM((2, 2 * L, DSTATE), jnp.float32),
            pltpu.VMEM((2, 2, L, DIM), jnp.float32),
        ],
    )

    r3 = lambda a: a.reshape(E, 1, DIM)
    mixed = pl.pallas_call(
        _expert_body,
        grid_spec=grid_spec,
        out_shape=jax.ShapeDtypeStruct((B, 1, DIM), jnp.float32),
        compiler_params=pltpu.CompilerParams(
            dimension_semantics=("arbitrary",),
        ),
    )(sidx, sval, x, Wx, Wdt, r3(bdt), A_log, r3(Dexp),
      r3(g1), r3(b1), r3(g2), r3(b2))

    return mixed.reshape(B, DIM), aux[0, 0]
